# baseline (speedup 1.0000x reference)
import functools

import jax
import jax.numpy as jnp
from jax import lax
from jax.experimental import pallas as pl
from jax.experimental.pallas import tpu as pltpu
from jax.experimental.pallas import tpu_sc as plsc

B, T, D = 2, 2048, 768
E = 8
TOPK = 2
DFF = 4 * D
N = B * T
C = N // E
NW = 32
DH = D // 2
KD = 512
RPT = N // 16


def _sc_gather(table, idx, nrows=N):
    rpw = nrows // NW
    nch = max(rpw // 128, 1)
    cw = rpw // nch
    mesh = plsc.VectorSubcoreMesh(core_axis_name="c", subcore_axis_name="s")

    @functools.partial(
        pl.kernel,
        mesh=mesh,
        out_type=jax.ShapeDtypeStruct((nrows, D), jnp.float32),
        scratch_types=[
            pltpu.VMEM((nch, cw), jnp.int32),
            pltpu.VMEM((cw, D), jnp.float32),
            pltpu.SemaphoreType.DMA,
        ],
    )
    def k(table_hbm, idx_hbm, out_hbm, idx_v, rows_v, sem):
        wid = lax.axis_index("s") * 2 + lax.axis_index("c")
        base = wid * rpw
        pltpu.sync_copy(idx_hbm.at[wid], idx_v)
        for q in range(nch):
            pltpu.async_copy(table_hbm.at[idx_v.at[q]], rows_v, sem).wait()
            pltpu.sync_copy(rows_v,
                            out_hbm.at[pl.ds(base + q * cw, cw)])

    return k(table, idx.reshape(NW, nch, cw))


NV = N // 16
H1, H2 = 2048, 1024


def _sc_select(pT_bits):
    mesh = plsc.VectorSubcoreMesh(core_axis_name="c", subcore_axis_name="s")

    @functools.partial(
        pl.kernel,
        mesh=mesh,
        compiler_params=pltpu.CompilerParams(needs_layout_passes=False),
        out_type=(jax.ShapeDtypeStruct((E, 16), jnp.float32),
                  jax.ShapeDtypeStruct((E, 16), jnp.int32)),
        scratch_types=[
            pltpu.VMEM((N,), jnp.int32),
            pltpu.VMEM((H1,), jnp.int32),
            pltpu.VMEM((16,), jnp.float32),
            pltpu.VMEM((16,), jnp.int32),
            pltpu.SemaphoreType.DMA,
        ],
    )
    def k(p_hbm, t_hbm, g_hbm, k_v, hist_v, t_v, g_v, sem):
        core = lax.axis_index("c")
        s = lax.axis_index("s")

        @pl.when((core == 0) & (s < E))
        def _():
            pltpu.sync_copy(p_hbm.at[s], k_v)
            iota = lax.iota(jnp.int32, 16)
            ones = jnp.ones((16,), jnp.int32)

            def hist_pass(shift, mask_val, nbuckets):
                @plsc.parallel_loop(0, nbuckets // 16)
                def _z(i):
                    hist_v[pl.ds(i * 16, 16)] = jnp.zeros((16,), jnp.int32)

                @plsc.parallel_loop(0, NV)
                def _h(i):
                    kv = k_v[pl.ds(i * 16, 16)]
                    b = lax.shift_right_logical(kv, shift)
                    if mask_val is None:
                        bb = jnp.minimum(b, nbuckets - 1)
                        plsc.addupdate_scatter(hist_v.at[...], [bb], ones)
                    else:
                        bb = jnp.bitwise_and(b, nbuckets - 1)
                        hi = lax.shift_right_logical(kv, shift + 10)
                        plsc.addupdate_scatter(hist_v.at[...], [bb], ones,
                                               mask=hi == mask_val)

            def find(nbuckets, R):
                def body(i, carry):
                    acc, bkt, cab, found = carry
                    idx = nbuckets // 16 - 1 - i
                    v = hist_v[pl.ds(idx * 16, 16)]
                    rc = plsc.cumsum(lax.rev(v, (0,)))
                    tot = jnp.sum(rc * (iota == 15))
                    m = (acc + rc) >= R
                    hasx = jnp.sum(jnp.where(m, 1, 0))
                    l = jnp.sum(plsc.all_reduce_ffs(m) * (iota == 0))
                    rc_l = jnp.sum(rc * (iota == l))
                    v_at = jnp.sum(v * (iota == (15 - l)))
                    hit = (hasx > 0) & jnp.logical_not(found)
                    bkt = jnp.where(hit, idx * 16 + 15 - l, bkt)
                    cab = jnp.where(hit, acc + rc_l - v_at, cab)
                    return acc + tot, bkt, cab, found | (hasx > 0)

                _, bkt, cab, _ = lax.fori_loop(
                    0, nbuckets // 16, body,
                    (jnp.int32(0), jnp.int32(0), jnp.int32(0), False))
                return bkt, cab

            hist_pass(20, None, H1)
            b1, ca1 = find(H1, jnp.int32(C))
            hist_pass(10, b1, H2)
            b2, ca2 = find(H2, C - ca1)
            hist_pass(0, (b1 << 10) | b2, H2)
            b3, ca3 = find(H2, C - ca1 - ca2)
            tbits = (b1 << 20) | (b2 << 10) | b3
            g = ca1 + ca2 + ca3
            t_v[...] = plsc.bitcast(jnp.broadcast_to(tbits, (16,)),
                                    jnp.float32)
            g_v[...] = jnp.broadcast_to(g, (16,))
            pltpu.sync_copy(t_v, t_hbm.at[s])
            pltpu.sync_copy(g_v, g_hbm.at[s])

    return k(pT_bits)


def _gelu_exact(h):
    return 0.5 * h * (1.0 + lax.erf(h / 1.4142135623730951))


def _tc_mlp(routed, fc_w, proj_w):
    grid = (E, DFF // KD)

    def body(r_ref, fc_ref, pj_ref, out_ref):
        kk = pl.program_id(1)
        a = r_ref[...].astype(jnp.bfloat16)
        fw = fc_ref[0].astype(jnp.bfloat16)
        h = lax.dot_general(a, fw, (((1,), (1,)), ((), ())),
                            preferred_element_type=jnp.float32)
        h = _gelu_exact(h)
        pw = pj_ref[0].astype(jnp.bfloat16)
        contrib = lax.dot_general(h.astype(jnp.bfloat16), pw,
                                  (((1,), (1,)), ((), ())),
                                  preferred_element_type=jnp.float32)

        @pl.when(kk == 0)
        def _():
            out_ref[...] = contrib

        @pl.when(kk > 0)
        def _():
            out_ref[...] += contrib

    return pl.pallas_call(
        body,
        grid=grid,
        in_specs=[
            pl.BlockSpec((C, D), lambda e, k: (e, 0)),
            pl.BlockSpec((1, KD, D), lambda e, k: (e, k, 0)),
            pl.BlockSpec((1, D, KD), lambda e, k: (e, 0, k)),
        ],
        out_specs=pl.BlockSpec((C, D), lambda e, k: (e, 0)),
        out_shape=jax.ShapeDtypeStruct((N, D), jnp.float32),
    )(routed, fc_w, proj_w)


def _tc_combine(gA, gB, wA, wB):
    def body(ga_ref, gb_ref, wa_ref, wb_ref, out_ref):
        out_ref[...] = ga_ref[...] * wa_ref[...] + gb_ref[...] * wb_ref[...]

    blk = 512
    return pl.pallas_call(
        body,
        grid=(N // blk,),
        in_specs=[
            pl.BlockSpec((blk, D), lambda i: (i, 0)),
            pl.BlockSpec((blk, D), lambda i: (i + N // blk, 0)),
            pl.BlockSpec((blk, 1), lambda i: (i, 0)),
            pl.BlockSpec((blk, 1), lambda i: (i, 0)),
        ],
        out_specs=pl.BlockSpec((blk, D), lambda i: (i, 0)),
        out_shape=jax.ShapeDtypeStruct((N, D), jnp.float32),
    )(gA, gB, wA, wB)


CHK = 16


def _sc_combine(eo, slotAB, wAB):
    bpw = N // NW
    nch = bpw // CHK
    mesh = plsc.VectorSubcoreMesh(core_axis_name="c", subcore_axis_name="s")

    @functools.partial(
        pl.kernel,
        mesh=mesh,
        compiler_params=pltpu.CompilerParams(needs_layout_passes=False),
        out_type=jax.ShapeDtypeStruct((N, D), jnp.float32),
        scratch_types=[
            pltpu.VMEM((2, CHK, D), jnp.float32),
            pltpu.VMEM((2, CHK, D), jnp.float32),
            pltpu.VMEM((2, CHK, D), jnp.float32),
            pltpu.VMEM((2, nch, CHK), jnp.int32),
            pltpu.VMEM((2, bpw, 16), jnp.float32),
            pltpu.SemaphoreType.DMA,
            pltpu.SemaphoreType.DMA,
            pltpu.SemaphoreType.DMA,
        ],
    )
    def k(eo_hbm, slot_hbm, w_hbm, out_hbm, bufa, bufb, bufo, s_v, w_v,
          sga, sgb, sgo):
        wid = lax.axis_index("s") * 2 + lax.axis_index("c")
        base = wid * bpw
        pltpu.sync_copy(slot_hbm.at[wid], s_v)
        pltpu.sync_copy(w_hbm.at[wid], w_v)
        ha, hb, ho = [None, None], [None, None], [None, None]
        ha[0] = pltpu.async_copy(eo_hbm.at[s_v.at[0, 0]], bufa.at[0], sga)
        hb[0] = pltpu.async_copy(eo_hbm.at[s_v.at[1, 0]], bufb.at[0], sgb)
        for q in range(nch):
            sl = q % 2
            if q + 1 < nch:
                nsl = (q + 1) % 2
                ha[nsl] = pltpu.async_copy(
                    eo_hbm.at[s_v.at[0, q + 1]], bufa.at[nsl], sga)
                hb[nsl] = pltpu.async_copy(
                    eo_hbm.at[s_v.at[1, q + 1]], bufb.at[nsl], sgb)
            ha[sl].wait()
            hb[sl].wait()
            if q >= 2:
                ho[sl].wait()

            @plsc.parallel_loop(0, CHK)
            def _row(r, q=q, sl=sl):
                wa = w_v[0, q * CHK + r]
                wb = w_v[1, q * CHK + r]
                for j in range(D // 16):
                    s_ = pl.ds(j * 16, 16)
                    bufo[sl, r, s_] = bufa[sl, r, s_] * wa + bufb[sl, r, s_] * wb
            ho[sl] = pltpu.async_copy(
                bufo.at[sl], out_hbm.at[pl.ds(base + q * CHK, CHK)], sgo)
        ho[(nch - 1) % 2].wait()
        ho[(nch - 2) % 2].wait()

    return k(eo, slotAB, wAB)


def kernel(x, gate_w, gate_b, fc_w, proj_w):
    flat = x.reshape(N, D)
    logits = flat @ gate_w.T + gate_b
    topv, topi = lax.top_k(logits, TOPK)
    rows = jnp.arange(N)[:, None]
    sparse = jnp.full_like(logits, -jnp.inf).at[rows, topi].set(topv)
    probs = jax.nn.softmax(sparse, axis=-1)
    pT = probs.T
    masked = jnp.where(pT > 0, pT, -jnp.inf)
    _, sel = lax.top_k(masked, C)
    tgt = sel.reshape(N).astype(jnp.int32)
    slotmap = jnp.full((E, N), -1, jnp.int32).at[
        jnp.arange(E)[:, None], sel].set(
        (jnp.arange(E)[:, None] * C + jnp.arange(C)[None, :]).astype(jnp.int32))
    tok = jnp.arange(N)
    sA = slotmap[topi[:, 0], tok]
    sB = slotmap[topi[:, 1], tok]
    pk = jnp.take_along_axis(probs, topi, axis=1)
    wA = jnp.where(sA >= 0, pk[:, 0], 0.0)[:, None]
    wB = jnp.where(sB >= 0, pk[:, 1], 0.0)[:, None]
    routed = _sc_gather(flat, tgt)
    eo = _tc_mlp(routed, fc_w, proj_w)
    catAB = jnp.concatenate([jnp.where(sA >= 0, sA, tok),
                             jnp.where(sB >= 0, sB, tok)]).astype(jnp.int32)
    gAB = _sc_gather(eo, catAB, nrows=2 * N)
    out = _tc_combine(gAB, gAB, wA, wB)
    return out.reshape(B, T, D)


# speedup vs baseline: 1.0889x; 1.0889x over previous
import functools

import jax
import jax.numpy as jnp
from jax import lax
from jax.experimental import pallas as pl
from jax.experimental.pallas import tpu as pltpu
from jax.experimental.pallas import tpu_sc as plsc

B, T, D = 2, 2048, 768
E = 8
TOPK = 2
DFF = 4 * D
N = B * T
C = N // E
NW = 32
DH = D // 2
KD = 1024
RPT = N // 16


def _sc_gather(table, idx, nrows=N):
    rpw = nrows // NW
    nch = max(rpw // 128, 1)
    cw = rpw // nch
    mesh = plsc.VectorSubcoreMesh(core_axis_name="c", subcore_axis_name="s")

    @functools.partial(
        pl.kernel,
        mesh=mesh,
        out_type=jax.ShapeDtypeStruct((nrows, D), jnp.float32),
        scratch_types=[
            pltpu.VMEM((nch, cw), jnp.int32),
            pltpu.VMEM((cw, D), jnp.float32),
            pltpu.SemaphoreType.DMA,
        ],
    )
    def k(table_hbm, idx_hbm, out_hbm, idx_v, rows_v, sem):
        wid = lax.axis_index("s") * 2 + lax.axis_index("c")
        base = wid * rpw
        pltpu.sync_copy(idx_hbm.at[wid], idx_v)
        for q in range(nch):
            pltpu.async_copy(table_hbm.at[idx_v.at[q]], rows_v, sem).wait()
            pltpu.sync_copy(rows_v,
                            out_hbm.at[pl.ds(base + q * cw, cw)])

    return k(table, idx.reshape(NW, nch, cw))


NV = N // 16
H1, H2 = 2048, 1024


def _sc_select(pT_bits):
    mesh = plsc.VectorSubcoreMesh(core_axis_name="c", subcore_axis_name="s")

    @functools.partial(
        pl.kernel,
        mesh=mesh,
        compiler_params=pltpu.CompilerParams(needs_layout_passes=False),
        out_type=(jax.ShapeDtypeStruct((E, 16), jnp.float32),
                  jax.ShapeDtypeStruct((E, 16), jnp.int32)),
        scratch_types=[
            pltpu.VMEM((N,), jnp.int32),
            pltpu.VMEM((H1,), jnp.int32),
            pltpu.VMEM((16,), jnp.float32),
            pltpu.VMEM((16,), jnp.int32),
            pltpu.SemaphoreType.DMA,
        ],
    )
    def k(p_hbm, t_hbm, g_hbm, k_v, hist_v, t_v, g_v, sem):
        core = lax.axis_index("c")
        s = lax.axis_index("s")

        @pl.when((core == 0) & (s < E))
        def _():
            pltpu.sync_copy(p_hbm.at[s], k_v)
            iota = lax.iota(jnp.int32, 16)
            ones = jnp.ones((16,), jnp.int32)

            def hist_pass(shift, mask_val, nbuckets):
                @plsc.parallel_loop(0, nbuckets // 16)
                def _z(i):
                    hist_v[pl.ds(i * 16, 16)] = jnp.zeros((16,), jnp.int32)

                @plsc.parallel_loop(0, NV)
                def _h(i):
                    kv = k_v[pl.ds(i * 16, 16)]
                    b = lax.shift_right_logical(kv, shift)
                    if mask_val is None:
                        bb = jnp.minimum(b, nbuckets - 1)
                        plsc.addupdate_scatter(hist_v.at[...], [bb], ones)
                    else:
                        bb = jnp.bitwise_and(b, nbuckets - 1)
                        hi = lax.shift_right_logical(kv, shift + 10)
                        plsc.addupdate_scatter(hist_v.at[...], [bb], ones,
                                               mask=hi == mask_val)

            def find(nbuckets, R):
                def body(i, carry):
                    acc, bkt, cab, found = carry
                    idx = nbuckets // 16 - 1 - i
                    v = hist_v[pl.ds(idx * 16, 16)]
                    rc = plsc.cumsum(lax.rev(v, (0,)))
                    tot = jnp.sum(rc * (iota == 15))
                    m = (acc + rc) >= R
                    hasx = jnp.sum(jnp.where(m, 1, 0))
                    l = jnp.sum(plsc.all_reduce_ffs(m) * (iota == 0))
                    rc_l = jnp.sum(rc * (iota == l))
                    v_at = jnp.sum(v * (iota == (15 - l)))
                    hit = (hasx > 0) & jnp.logical_not(found)
                    bkt = jnp.where(hit, idx * 16 + 15 - l, bkt)
                    cab = jnp.where(hit, acc + rc_l - v_at, cab)
                    return acc + tot, bkt, cab, found | (hasx > 0)

                _, bkt, cab, _ = lax.fori_loop(
                    0, nbuckets // 16, body,
                    (jnp.int32(0), jnp.int32(0), jnp.int32(0), False))
                return bkt, cab

            hist_pass(20, None, H1)
            b1, ca1 = find(H1, jnp.int32(C))
            hist_pass(10, b1, H2)
            b2, ca2 = find(H2, C - ca1)
            hist_pass(0, (b1 << 10) | b2, H2)
            b3, ca3 = find(H2, C - ca1 - ca2)
            tbits = (b1 << 20) | (b2 << 10) | b3
            g = ca1 + ca2 + ca3
            t_v[...] = plsc.bitcast(jnp.broadcast_to(tbits, (16,)),
                                    jnp.float32)
            g_v[...] = jnp.broadcast_to(g, (16,))
            pltpu.sync_copy(t_v, t_hbm.at[s])
            pltpu.sync_copy(g_v, g_hbm.at[s])

    return k(pT_bits)


def _gelu_exact(h):
    return 0.5 * h * (1.0 + lax.erf(h / 1.4142135623730951))


def _tc_mlp(routed, fc_w, proj_w):
    grid = (E, DFF // KD)

    def body(r_ref, fc_ref, pj_ref, out_ref):
        kk = pl.program_id(1)
        a = r_ref[...]
        fw = fc_ref[0]
        h = lax.dot_general(a, fw, (((1,), (1,)), ((), ())),
                            preferred_element_type=jnp.float32)
        h = _gelu_exact(h)
        pw = pj_ref[0]
        contrib = lax.dot_general(h, pw, (((1,), (1,)), ((), ())),
                                  preferred_element_type=jnp.float32)

        @pl.when(kk == 0)
        def _():
            out_ref[...] = contrib

        @pl.when(kk > 0)
        def _():
            out_ref[...] += contrib

    return pl.pallas_call(
        body,
        grid=grid,
        in_specs=[
            pl.BlockSpec((C, D), lambda e, k: (e, 0)),
            pl.BlockSpec((1, KD, D), lambda e, k: (e, k, 0)),
            pl.BlockSpec((1, D, KD), lambda e, k: (e, 0, k)),
        ],
        out_specs=pl.BlockSpec((C, D), lambda e, k: (e, 0)),
        out_shape=jax.ShapeDtypeStruct((N, D), jnp.float32),
    )(routed, fc_w, proj_w)


def _tc_combine(gA, gB, wA, wB):
    def body(ga_ref, gb_ref, wa_ref, wb_ref, out_ref):
        out_ref[...] = ga_ref[...] * wa_ref[...] + gb_ref[...] * wb_ref[...]

    blk = 512
    return pl.pallas_call(
        body,
        grid=(N // blk,),
        in_specs=[
            pl.BlockSpec((blk, D), lambda i: (i, 0)),
            pl.BlockSpec((blk, D), lambda i: (i + N // blk, 0)),
            pl.BlockSpec((blk, 1), lambda i: (i, 0)),
            pl.BlockSpec((blk, 1), lambda i: (i, 0)),
        ],
        out_specs=pl.BlockSpec((blk, D), lambda i: (i, 0)),
        out_shape=jax.ShapeDtypeStruct((N, D), jnp.float32),
    )(gA, gB, wA, wB)


CHK = 16


def _sc_combine(eo, slotAB, wAB):
    bpw = N // NW
    nch = bpw // CHK
    mesh = plsc.VectorSubcoreMesh(core_axis_name="c", subcore_axis_name="s")

    @functools.partial(
        pl.kernel,
        mesh=mesh,
        compiler_params=pltpu.CompilerParams(needs_layout_passes=False),
        out_type=jax.ShapeDtypeStruct((N, D), jnp.float32),
        scratch_types=[
            pltpu.VMEM((2, CHK, D), jnp.float32),
            pltpu.VMEM((2, CHK, D), jnp.float32),
            pltpu.VMEM((2, CHK, D), jnp.float32),
            pltpu.VMEM((2, nch, CHK), jnp.int32),
            pltpu.VMEM((2, bpw, 16), jnp.float32),
            pltpu.SemaphoreType.DMA,
            pltpu.SemaphoreType.DMA,
            pltpu.SemaphoreType.DMA,
        ],
    )
    def k(eo_hbm, slot_hbm, w_hbm, out_hbm, bufa, bufb, bufo, s_v, w_v,
          sga, sgb, sgo):
        wid = lax.axis_index("s") * 2 + lax.axis_index("c")
        base = wid * bpw
        pltpu.sync_copy(slot_hbm.at[wid], s_v)
        pltpu.sync_copy(w_hbm.at[wid], w_v)
        ha, hb, ho = [None, None], [None, None], [None, None]
        ha[0] = pltpu.async_copy(eo_hbm.at[s_v.at[0, 0]], bufa.at[0], sga)
        hb[0] = pltpu.async_copy(eo_hbm.at[s_v.at[1, 0]], bufb.at[0], sgb)
        for q in range(nch):
            sl = q % 2
            if q + 1 < nch:
                nsl = (q + 1) % 2
                ha[nsl] = pltpu.async_copy(
                    eo_hbm.at[s_v.at[0, q + 1]], bufa.at[nsl], sga)
                hb[nsl] = pltpu.async_copy(
                    eo_hbm.at[s_v.at[1, q + 1]], bufb.at[nsl], sgb)
            ha[sl].wait()
            hb[sl].wait()
            if q >= 2:
                ho[sl].wait()

            @plsc.parallel_loop(0, CHK)
            def _row(r, q=q, sl=sl):
                wa = w_v[0, q * CHK + r]
                wb = w_v[1, q * CHK + r]
                for j in range(D // 16):
                    s_ = pl.ds(j * 16, 16)
                    bufo[sl, r, s_] = bufa[sl, r, s_] * wa + bufb[sl, r, s_] * wb
            ho[sl] = pltpu.async_copy(
                bufo.at[sl], out_hbm.at[pl.ds(base + q * CHK, CHK)], sgo)
        ho[(nch - 1) % 2].wait()
        ho[(nch - 2) % 2].wait()

    return k(eo, slotAB, wAB)


def kernel(x, gate_w, gate_b, fc_w, proj_w):
    flat = x.reshape(N, D)
    logits = flat @ gate_w.T + gate_b
    topv, topi = lax.top_k(logits, TOPK)
    rows = jnp.arange(N)[:, None]
    sparse = jnp.full_like(logits, -jnp.inf).at[rows, topi].set(topv)
    probs = jax.nn.softmax(sparse, axis=-1)
    pT = probs.T
    masked = jnp.where(pT > 0, pT, -jnp.inf)
    _, sel = lax.top_k(masked, C)
    tgt = sel.reshape(N).astype(jnp.int32)
    slotmap = jnp.full((E, N), -1, jnp.int32).at[
        jnp.arange(E)[:, None], sel].set(
        (jnp.arange(E)[:, None] * C + jnp.arange(C)[None, :]).astype(jnp.int32))
    tok = jnp.arange(N)
    sA = slotmap[topi[:, 0], tok]
    sB = slotmap[topi[:, 1], tok]
    pk = jnp.take_along_axis(probs, topi, axis=1)
    wA = jnp.where(sA >= 0, pk[:, 0], 0.0)[:, None]
    wB = jnp.where(sB >= 0, pk[:, 1], 0.0)[:, None]
    routed = _sc_gather(flat, tgt)
    eo = _tc_mlp(routed, fc_w, proj_w)
    catAB = jnp.concatenate([jnp.where(sA >= 0, sA, tok),
                             jnp.where(sB >= 0, sB, tok)]).astype(jnp.int32)
    gAB = _sc_gather(eo, catAB, nrows=2 * N)
    out = _tc_combine(gAB, gAB, wA, wB)
    return out.reshape(B, T, D)


# speedup vs baseline: 1.1156x; 1.0245x over previous
import functools

import jax
import jax.numpy as jnp
from jax import lax
from jax.experimental import pallas as pl
from jax.experimental.pallas import tpu as pltpu
from jax.experimental.pallas import tpu_sc as plsc

B, T, D = 2, 2048, 768
E = 8
TOPK = 2
DFF = 4 * D
N = B * T
C = N // E
NW = 32
DH = D // 2
KD = 1536
RPT = N // 16


def _sc_gather(table, idx, nrows=N):
    rpw = nrows // NW
    nch = max(rpw // 128, 1)
    cw = rpw // nch
    mesh = plsc.VectorSubcoreMesh(core_axis_name="c", subcore_axis_name="s")

    @functools.partial(
        pl.kernel,
        mesh=mesh,
        out_type=jax.ShapeDtypeStruct((nrows, D), jnp.float32),
        scratch_types=[
            pltpu.VMEM((nch, cw), jnp.int32),
            pltpu.VMEM((cw, D), jnp.float32),
            pltpu.SemaphoreType.DMA,
        ],
    )
    def k(table_hbm, idx_hbm, out_hbm, idx_v, rows_v, sem):
        wid = lax.axis_index("s") * 2 + lax.axis_index("c")
        base = wid * rpw
        pltpu.sync_copy(idx_hbm.at[wid], idx_v)
        for q in range(nch):
            pltpu.async_copy(table_hbm.at[idx_v.at[q]], rows_v, sem).wait()
            pltpu.sync_copy(rows_v,
                            out_hbm.at[pl.ds(base + q * cw, cw)])

    return k(table, idx.reshape(NW, nch, cw))


NV = N // 16
H1, H2 = 2048, 1024


def _sc_select(pT_bits):
    mesh = plsc.VectorSubcoreMesh(core_axis_name="c", subcore_axis_name="s")

    @functools.partial(
        pl.kernel,
        mesh=mesh,
        compiler_params=pltpu.CompilerParams(needs_layout_passes=False),
        out_type=(jax.ShapeDtypeStruct((E, 16), jnp.float32),
                  jax.ShapeDtypeStruct((E, 16), jnp.int32)),
        scratch_types=[
            pltpu.VMEM((N,), jnp.int32),
            pltpu.VMEM((H1,), jnp.int32),
            pltpu.VMEM((16,), jnp.float32),
            pltpu.VMEM((16,), jnp.int32),
            pltpu.SemaphoreType.DMA,
        ],
    )
    def k(p_hbm, t_hbm, g_hbm, k_v, hist_v, t_v, g_v, sem):
        core = lax.axis_index("c")
        s = lax.axis_index("s")

        @pl.when((core == 0) & (s < E))
        def _():
            pltpu.sync_copy(p_hbm.at[s], k_v)
            iota = lax.iota(jnp.int32, 16)
            ones = jnp.ones((16,), jnp.int32)

            def hist_pass(shift, mask_val, nbuckets):
                @plsc.parallel_loop(0, nbuckets // 16)
                def _z(i):
                    hist_v[pl.ds(i * 16, 16)] = jnp.zeros((16,), jnp.int32)

                @plsc.parallel_loop(0, NV)
                def _h(i):
                    kv = k_v[pl.ds(i * 16, 16)]
                    b = lax.shift_right_logical(kv, shift)
                    if mask_val is None:
                        bb = jnp.minimum(b, nbuckets - 1)
                        plsc.addupdate_scatter(hist_v.at[...], [bb], ones)
                    else:
                        bb = jnp.bitwise_and(b, nbuckets - 1)
                        hi = lax.shift_right_logical(kv, shift + 10)
                        plsc.addupdate_scatter(hist_v.at[...], [bb], ones,
                                               mask=hi == mask_val)

            def find(nbuckets, R):
                def body(i, carry):
                    acc, bkt, cab, found = carry
                    idx = nbuckets // 16 - 1 - i
                    v = hist_v[pl.ds(idx * 16, 16)]
                    rc = plsc.cumsum(lax.rev(v, (0,)))
                    tot = jnp.sum(rc * (iota == 15))
                    m = (acc + rc) >= R
                    hasx = jnp.sum(jnp.where(m, 1, 0))
                    l = jnp.sum(plsc.all_reduce_ffs(m) * (iota == 0))
                    rc_l = jnp.sum(rc * (iota == l))
                    v_at = jnp.sum(v * (iota == (15 - l)))
                    hit = (hasx > 0) & jnp.logical_not(found)
                    bkt = jnp.where(hit, idx * 16 + 15 - l, bkt)
                    cab = jnp.where(hit, acc + rc_l - v_at, cab)
                    return acc + tot, bkt, cab, found | (hasx > 0)

                _, bkt, cab, _ = lax.fori_loop(
                    0, nbuckets // 16, body,
                    (jnp.int32(0), jnp.int32(0), jnp.int32(0), False))
                return bkt, cab

            hist_pass(20, None, H1)
            b1, ca1 = find(H1, jnp.int32(C))
            hist_pass(10, b1, H2)
            b2, ca2 = find(H2, C - ca1)
            hist_pass(0, (b1 << 10) | b2, H2)
            b3, ca3 = find(H2, C - ca1 - ca2)
            tbits = (b1 << 20) | (b2 << 10) | b3
            g = ca1 + ca2 + ca3
            t_v[...] = plsc.bitcast(jnp.broadcast_to(tbits, (16,)),
                                    jnp.float32)
            g_v[...] = jnp.broadcast_to(g, (16,))
            pltpu.sync_copy(t_v, t_hbm.at[s])
            pltpu.sync_copy(g_v, g_hbm.at[s])

    return k(pT_bits)


def _gelu_exact(h):
    return 0.5 * h * (1.0 + lax.erf(h / 1.4142135623730951))


def _tc_mlp(routed, fc_w, proj_w):
    grid = (E, DFF // KD)

    def body(r_ref, fc_ref, pj_ref, out_ref):
        kk = pl.program_id(1)
        a = r_ref[...]
        fw = fc_ref[0]
        h = lax.dot_general(a, fw, (((1,), (1,)), ((), ())),
                            preferred_element_type=jnp.float32)
        h = _gelu_exact(h)
        pw = pj_ref[0]
        contrib = lax.dot_general(h, pw, (((1,), (1,)), ((), ())),
                                  preferred_element_type=jnp.float32)

        @pl.when(kk == 0)
        def _():
            out_ref[...] = contrib

        @pl.when(kk > 0)
        def _():
            out_ref[...] += contrib

    return pl.pallas_call(
        body,
        grid=grid,
        in_specs=[
            pl.BlockSpec((C, D), lambda e, k: (e, 0)),
            pl.BlockSpec((1, KD, D), lambda e, k: (e, k, 0)),
            pl.BlockSpec((1, D, KD), lambda e, k: (e, 0, k)),
        ],
        out_specs=pl.BlockSpec((C, D), lambda e, k: (e, 0)),
        out_shape=jax.ShapeDtypeStruct((N, D), jnp.float32),
    )(routed, fc_w, proj_w)


def _tc_combine(gA, gB, wA, wB):
    def body(ga_ref, gb_ref, wa_ref, wb_ref, out_ref):
        out_ref[...] = ga_ref[...] * wa_ref[...] + gb_ref[...] * wb_ref[...]

    blk = 512
    return pl.pallas_call(
        body,
        grid=(N // blk,),
        in_specs=[
            pl.BlockSpec((blk, D), lambda i: (i, 0)),
            pl.BlockSpec((blk, D), lambda i: (i + N // blk, 0)),
            pl.BlockSpec((blk, 1), lambda i: (i, 0)),
            pl.BlockSpec((blk, 1), lambda i: (i, 0)),
        ],
        out_specs=pl.BlockSpec((blk, D), lambda i: (i, 0)),
        out_shape=jax.ShapeDtypeStruct((N, D), jnp.float32),
    )(gA, gB, wA, wB)


CHK = 16


def _sc_combine(eo, slotAB, wAB):
    bpw = N // NW
    nch = bpw // CHK
    mesh = plsc.VectorSubcoreMesh(core_axis_name="c", subcore_axis_name="s")

    @functools.partial(
        pl.kernel,
        mesh=mesh,
        compiler_params=pltpu.CompilerParams(needs_layout_passes=False),
        out_type=jax.ShapeDtypeStruct((N, D), jnp.float32),
        scratch_types=[
            pltpu.VMEM((2, CHK, D), jnp.float32),
            pltpu.VMEM((2, CHK, D), jnp.float32),
            pltpu.VMEM((2, CHK, D), jnp.float32),
            pltpu.VMEM((2, nch, CHK), jnp.int32),
            pltpu.VMEM((2, bpw, 16), jnp.float32),
            pltpu.SemaphoreType.DMA,
            pltpu.SemaphoreType.DMA,
            pltpu.SemaphoreType.DMA,
        ],
    )
    def k(eo_hbm, slot_hbm, w_hbm, out_hbm, bufa, bufb, bufo, s_v, w_v,
          sga, sgb, sgo):
        wid = lax.axis_index("s") * 2 + lax.axis_index("c")
        base = wid * bpw
        pltpu.sync_copy(slot_hbm.at[wid], s_v)
        pltpu.sync_copy(w_hbm.at[wid], w_v)
        ha, hb, ho = [None, None], [None, None], [None, None]
        ha[0] = pltpu.async_copy(eo_hbm.at[s_v.at[0, 0]], bufa.at[0], sga)
        hb[0] = pltpu.async_copy(eo_hbm.at[s_v.at[1, 0]], bufb.at[0], sgb)
        for q in range(nch):
            sl = q % 2
            if q + 1 < nch:
                nsl = (q + 1) % 2
                ha[nsl] = pltpu.async_copy(
                    eo_hbm.at[s_v.at[0, q + 1]], bufa.at[nsl], sga)
                hb[nsl] = pltpu.async_copy(
                    eo_hbm.at[s_v.at[1, q + 1]], bufb.at[nsl], sgb)
            ha[sl].wait()
            hb[sl].wait()
            if q >= 2:
                ho[sl].wait()

            @plsc.parallel_loop(0, CHK)
            def _row(r, q=q, sl=sl):
                wa = w_v[0, q * CHK + r]
                wb = w_v[1, q * CHK + r]
                for j in range(D // 16):
                    s_ = pl.ds(j * 16, 16)
                    bufo[sl, r, s_] = bufa[sl, r, s_] * wa + bufb[sl, r, s_] * wb
            ho[sl] = pltpu.async_copy(
                bufo.at[sl], out_hbm.at[pl.ds(base + q * CHK, CHK)], sgo)
        ho[(nch - 1) % 2].wait()
        ho[(nch - 2) % 2].wait()

    return k(eo, slotAB, wAB)


def kernel(x, gate_w, gate_b, fc_w, proj_w):
    flat = x.reshape(N, D)
    logits = flat @ gate_w.T + gate_b
    topv, topi = lax.top_k(logits, TOPK)
    rows = jnp.arange(N)[:, None]
    sparse = jnp.full_like(logits, -jnp.inf).at[rows, topi].set(topv)
    probs = jax.nn.softmax(sparse, axis=-1)
    pT = probs.T
    masked = jnp.where(pT > 0, pT, -jnp.inf)
    _, sel = lax.top_k(masked, C)
    tgt = sel.reshape(N).astype(jnp.int32)
    slotmap = jnp.full((E, N), -1, jnp.int32).at[
        jnp.arange(E)[:, None], sel].set(
        (jnp.arange(E)[:, None] * C + jnp.arange(C)[None, :]).astype(jnp.int32))
    tok = jnp.arange(N)
    sA = slotmap[topi[:, 0], tok]
    sB = slotmap[topi[:, 1], tok]
    pk = jnp.take_along_axis(probs, topi, axis=1)
    wA = jnp.where(sA >= 0, pk[:, 0], 0.0)[:, None]
    wB = jnp.where(sB >= 0, pk[:, 1], 0.0)[:, None]
    routed = _sc_gather(flat, tgt)
    eo = _tc_mlp(routed, fc_w, proj_w)
    catAB = jnp.concatenate([jnp.where(sA >= 0, sA, tok),
                             jnp.where(sB >= 0, sB, tok)]).astype(jnp.int32)
    gAB = _sc_gather(eo, catAB, nrows=2 * N)
    out = _tc_combine(gAB, gAB, wA, wB)
    return out.reshape(B, T, D)


# speedup vs baseline: 1.1380x; 1.0201x over previous
import functools

import jax
import jax.numpy as jnp
from jax import lax
from jax.experimental import pallas as pl
from jax.experimental.pallas import tpu as pltpu
from jax.experimental.pallas import tpu_sc as plsc

B, T, D = 2, 2048, 768
E = 8
TOPK = 2
DFF = 4 * D
N = B * T
C = N // E
NW = 32
DH = D // 2
KD = 3072
RPT = N // 16


def _sc_gather(table, idx, nrows=N):
    rpw = nrows // NW
    nch = max(rpw // 128, 1)
    cw = rpw // nch
    mesh = plsc.VectorSubcoreMesh(core_axis_name="c", subcore_axis_name="s")

    @functools.partial(
        pl.kernel,
        mesh=mesh,
        out_type=jax.ShapeDtypeStruct((nrows, D), jnp.float32),
        scratch_types=[
            pltpu.VMEM((nch, cw), jnp.int32),
            pltpu.VMEM((cw, D), jnp.float32),
            pltpu.SemaphoreType.DMA,
        ],
    )
    def k(table_hbm, idx_hbm, out_hbm, idx_v, rows_v, sem):
        wid = lax.axis_index("s") * 2 + lax.axis_index("c")
        base = wid * rpw
        pltpu.sync_copy(idx_hbm.at[wid], idx_v)
        for q in range(nch):
            pltpu.async_copy(table_hbm.at[idx_v.at[q]], rows_v, sem).wait()
            pltpu.sync_copy(rows_v,
                            out_hbm.at[pl.ds(base + q * cw, cw)])

    return k(table, idx.reshape(NW, nch, cw))


NV = N // 16
H1, H2 = 2048, 1024


def _sc_select(pT_bits):
    mesh = plsc.VectorSubcoreMesh(core_axis_name="c", subcore_axis_name="s")

    @functools.partial(
        pl.kernel,
        mesh=mesh,
        compiler_params=pltpu.CompilerParams(needs_layout_passes=False),
        out_type=(jax.ShapeDtypeStruct((E, 16), jnp.float32),
                  jax.ShapeDtypeStruct((E, 16), jnp.int32)),
        scratch_types=[
            pltpu.VMEM((N,), jnp.int32),
            pltpu.VMEM((H1,), jnp.int32),
            pltpu.VMEM((16,), jnp.float32),
            pltpu.VMEM((16,), jnp.int32),
            pltpu.SemaphoreType.DMA,
        ],
    )
    def k(p_hbm, t_hbm, g_hbm, k_v, hist_v, t_v, g_v, sem):
        core = lax.axis_index("c")
        s = lax.axis_index("s")

        @pl.when((core == 0) & (s < E))
        def _():
            pltpu.sync_copy(p_hbm.at[s], k_v)
            iota = lax.iota(jnp.int32, 16)
            ones = jnp.ones((16,), jnp.int32)

            def hist_pass(shift, mask_val, nbuckets):
                @plsc.parallel_loop(0, nbuckets // 16)
                def _z(i):
                    hist_v[pl.ds(i * 16, 16)] = jnp.zeros((16,), jnp.int32)

                @plsc.parallel_loop(0, NV)
                def _h(i):
                    kv = k_v[pl.ds(i * 16, 16)]
                    b = lax.shift_right_logical(kv, shift)
                    if mask_val is None:
                        bb = jnp.minimum(b, nbuckets - 1)
                        plsc.addupdate_scatter(hist_v.at[...], [bb], ones)
                    else:
                        bb = jnp.bitwise_and(b, nbuckets - 1)
                        hi = lax.shift_right_logical(kv, shift + 10)
                        plsc.addupdate_scatter(hist_v.at[...], [bb], ones,
                                               mask=hi == mask_val)

            def find(nbuckets, R):
                def body(i, carry):
                    acc, bkt, cab, found = carry
                    idx = nbuckets // 16 - 1 - i
                    v = hist_v[pl.ds(idx * 16, 16)]
                    rc = plsc.cumsum(lax.rev(v, (0,)))
                    tot = jnp.sum(rc * (iota == 15))
                    m = (acc + rc) >= R
                    hasx = jnp.sum(jnp.where(m, 1, 0))
                    l = jnp.sum(plsc.all_reduce_ffs(m) * (iota == 0))
                    rc_l = jnp.sum(rc * (iota == l))
                    v_at = jnp.sum(v * (iota == (15 - l)))
                    hit = (hasx > 0) & jnp.logical_not(found)
                    bkt = jnp.where(hit, idx * 16 + 15 - l, bkt)
                    cab = jnp.where(hit, acc + rc_l - v_at, cab)
                    return acc + tot, bkt, cab, found | (hasx > 0)

                _, bkt, cab, _ = lax.fori_loop(
                    0, nbuckets // 16, body,
                    (jnp.int32(0), jnp.int32(0), jnp.int32(0), False))
                return bkt, cab

            hist_pass(20, None, H1)
            b1, ca1 = find(H1, jnp.int32(C))
            hist_pass(10, b1, H2)
            b2, ca2 = find(H2, C - ca1)
            hist_pass(0, (b1 << 10) | b2, H2)
            b3, ca3 = find(H2, C - ca1 - ca2)
            tbits = (b1 << 20) | (b2 << 10) | b3
            g = ca1 + ca2 + ca3
            t_v[...] = plsc.bitcast(jnp.broadcast_to(tbits, (16,)),
                                    jnp.float32)
            g_v[...] = jnp.broadcast_to(g, (16,))
            pltpu.sync_copy(t_v, t_hbm.at[s])
            pltpu.sync_copy(g_v, g_hbm.at[s])

    return k(pT_bits)


def _gelu_exact(h):
    return 0.5 * h * (1.0 + lax.erf(h / 1.4142135623730951))


def _tc_mlp(routed, fc_w, proj_w):
    grid = (E, DFF // KD)

    def body(r_ref, fc_ref, pj_ref, out_ref):
        kk = pl.program_id(1)
        a = r_ref[...]
        fw = fc_ref[0]
        h = lax.dot_general(a, fw, (((1,), (1,)), ((), ())),
                            preferred_element_type=jnp.float32)
        h = _gelu_exact(h)
        pw = pj_ref[0]
        contrib = lax.dot_general(h, pw, (((1,), (1,)), ((), ())),
                                  preferred_element_type=jnp.float32)

        @pl.when(kk == 0)
        def _():
            out_ref[...] = contrib

        @pl.when(kk > 0)
        def _():
            out_ref[...] += contrib

    return pl.pallas_call(
        body,
        grid=grid,
        in_specs=[
            pl.BlockSpec((C, D), lambda e, k: (e, 0)),
            pl.BlockSpec((1, KD, D), lambda e, k: (e, k, 0)),
            pl.BlockSpec((1, D, KD), lambda e, k: (e, 0, k)),
        ],
        out_specs=pl.BlockSpec((C, D), lambda e, k: (e, 0)),
        out_shape=jax.ShapeDtypeStruct((N, D), jnp.float32),
    )(routed, fc_w, proj_w)


def _tc_combine(gA, gB, wA, wB):
    def body(ga_ref, gb_ref, wa_ref, wb_ref, out_ref):
        out_ref[...] = ga_ref[...] * wa_ref[...] + gb_ref[...] * wb_ref[...]

    blk = 512
    return pl.pallas_call(
        body,
        grid=(N // blk,),
        in_specs=[
            pl.BlockSpec((blk, D), lambda i: (i, 0)),
            pl.BlockSpec((blk, D), lambda i: (i + N // blk, 0)),
            pl.BlockSpec((blk, 1), lambda i: (i, 0)),
            pl.BlockSpec((blk, 1), lambda i: (i, 0)),
        ],
        out_specs=pl.BlockSpec((blk, D), lambda i: (i, 0)),
        out_shape=jax.ShapeDtypeStruct((N, D), jnp.float32),
    )(gA, gB, wA, wB)


CHK = 16


def _sc_combine(eo, slotAB, wAB):
    bpw = N // NW
    nch = bpw // CHK
    mesh = plsc.VectorSubcoreMesh(core_axis_name="c", subcore_axis_name="s")

    @functools.partial(
        pl.kernel,
        mesh=mesh,
        compiler_params=pltpu.CompilerParams(needs_layout_passes=False),
        out_type=jax.ShapeDtypeStruct((N, D), jnp.float32),
        scratch_types=[
            pltpu.VMEM((2, CHK, D), jnp.float32),
            pltpu.VMEM((2, CHK, D), jnp.float32),
            pltpu.VMEM((2, CHK, D), jnp.float32),
            pltpu.VMEM((2, nch, CHK), jnp.int32),
            pltpu.VMEM((2, bpw, 16), jnp.float32),
            pltpu.SemaphoreType.DMA,
            pltpu.SemaphoreType.DMA,
            pltpu.SemaphoreType.DMA,
        ],
    )
    def k(eo_hbm, slot_hbm, w_hbm, out_hbm, bufa, bufb, bufo, s_v, w_v,
          sga, sgb, sgo):
        wid = lax.axis_index("s") * 2 + lax.axis_index("c")
        base = wid * bpw
        pltpu.sync_copy(slot_hbm.at[wid], s_v)
        pltpu.sync_copy(w_hbm.at[wid], w_v)
        ha, hb, ho = [None, None], [None, None], [None, None]
        ha[0] = pltpu.async_copy(eo_hbm.at[s_v.at[0, 0]], bufa.at[0], sga)
        hb[0] = pltpu.async_copy(eo_hbm.at[s_v.at[1, 0]], bufb.at[0], sgb)
        for q in range(nch):
            sl = q % 2
            if q + 1 < nch:
                nsl = (q + 1) % 2
                ha[nsl] = pltpu.async_copy(
                    eo_hbm.at[s_v.at[0, q + 1]], bufa.at[nsl], sga)
                hb[nsl] = pltpu.async_copy(
                    eo_hbm.at[s_v.at[1, q + 1]], bufb.at[nsl], sgb)
            ha[sl].wait()
            hb[sl].wait()
            if q >= 2:
                ho[sl].wait()

            @plsc.parallel_loop(0, CHK)
            def _row(r, q=q, sl=sl):
                wa = w_v[0, q * CHK + r]
                wb = w_v[1, q * CHK + r]
                for j in range(D // 16):
                    s_ = pl.ds(j * 16, 16)
                    bufo[sl, r, s_] = bufa[sl, r, s_] * wa + bufb[sl, r, s_] * wb
            ho[sl] = pltpu.async_copy(
                bufo.at[sl], out_hbm.at[pl.ds(base + q * CHK, CHK)], sgo)
        ho[(nch - 1) % 2].wait()
        ho[(nch - 2) % 2].wait()

    return k(eo, slotAB, wAB)


def kernel(x, gate_w, gate_b, fc_w, proj_w):
    flat = x.reshape(N, D)
    logits = flat @ gate_w.T + gate_b
    topv, topi = lax.top_k(logits, TOPK)
    rows = jnp.arange(N)[:, None]
    sparse = jnp.full_like(logits, -jnp.inf).at[rows, topi].set(topv)
    probs = jax.nn.softmax(sparse, axis=-1)
    pT = probs.T
    masked = jnp.where(pT > 0, pT, -jnp.inf)
    _, sel = lax.top_k(masked, C)
    tgt = sel.reshape(N).astype(jnp.int32)
    slotmap = jnp.full((E, N), -1, jnp.int32).at[
        jnp.arange(E)[:, None], sel].set(
        (jnp.arange(E)[:, None] * C + jnp.arange(C)[None, :]).astype(jnp.int32))
    tok = jnp.arange(N)
    sA = slotmap[topi[:, 0], tok]
    sB = slotmap[topi[:, 1], tok]
    pk = jnp.take_along_axis(probs, topi, axis=1)
    wA = jnp.where(sA >= 0, pk[:, 0], 0.0)[:, None]
    wB = jnp.where(sB >= 0, pk[:, 1], 0.0)[:, None]
    routed = _sc_gather(flat, tgt)
    eo = _tc_mlp(routed, fc_w, proj_w)
    catAB = jnp.concatenate([jnp.where(sA >= 0, sA, tok),
                             jnp.where(sB >= 0, sB, tok)]).astype(jnp.int32)
    gAB = _sc_gather(eo, catAB, nrows=2 * N)
    out = _tc_combine(gAB, gAB, wA, wB)
    return out.reshape(B, T, D)
